# baseline (speedup 1.0000x reference)
import functools

import jax
import jax.numpy as jnp
import numpy as np
from jax import lax
from jax.experimental import pallas as pl
from jax.experimental.pallas import tpu as pltpu
from jax.experimental.pallas import tpu_sc as plsc

_NC = 2
_NS = 16
_NW = _NC * _NS


def _sc_gather_all(forest0, forest1f, forest2f, fmp):
    B = forest0.shape[0]
    N1 = forest1f.shape[0]
    N2 = forest2f.shape[0]
    PW = fmp.shape[1]
    F = 2 * PW
    S2 = N2 // N1
    n0 = B // _NW
    n1 = N1 // _NW
    CH = 8
    ROWS = CH * S2
    nchunk1 = n1 // ROWS
    nseg = (N2 // S2) // _NW
    nchunk2 = nseg // CH

    mesh = plsc.VectorSubcoreMesh(core_axis_name="c", subcore_axis_name="s")

    @functools.partial(
        pl.kernel,
        mesh=mesh,
        out_type=[
            jax.ShapeDtypeStruct((B, PW), jnp.int32),
            jax.ShapeDtypeStruct((N1, PW), jnp.int32),
            jax.ShapeDtypeStruct((N1, F), jnp.float32),
        ],
        scratch_types=[
            pltpu.VMEM((n0,), jnp.int32),
            pltpu.VMEM((ROWS,), jnp.int32),
            pltpu.VMEM((ROWS,), jnp.int32),
            pltpu.VMEM((ROWS, PW), jnp.int32),
            pltpu.VMEM((ROWS, PW), jnp.int32),
            pltpu.VMEM((CH, F), jnp.float32),
            pltpu.SemaphoreType.DMA,
            pltpu.SemaphoreType.DMA,
        ],
    )
    def sc_kernel(f0_hbm, f1_hbm, f2_hbm, fm_hbm, out0, out1, out2,
                  idx0_v, idxa_v, idxb_v, bufa_v, bufb_v, acc_v, sema, semb):
        wid = lax.axis_index("s") * _NC + lax.axis_index("c")

        base0 = wid * n0
        pltpu.sync_copy(f0_hbm.at[pl.ds(base0, n0)], idx0_v)
        pltpu.async_copy(fm_hbm.at[idx0_v], bufa_v.at[pl.ds(0, n0)], sema).wait()
        pltpu.sync_copy(bufa_v.at[pl.ds(0, n0)], out0.at[pl.ds(base0, n0)])

        def f1_issue(c, idx_v, buf_v, sem):
            @pl.when(c < nchunk1)
            def _():
                base = wid * n1 + c * ROWS
                pltpu.sync_copy(f1_hbm.at[pl.ds(base, ROWS)], idx_v)
                pltpu.async_copy(fm_hbm.at[idx_v], buf_v, sem)

        def f1_drain(c, idx_v, buf_v, sem):
            pltpu.make_async_copy(fm_hbm.at[idx_v], buf_v, sem).wait()
            pltpu.sync_copy(buf_v, out1.at[pl.ds(wid * n1 + c * ROWS, ROWS)])

        f1_issue(0, idxa_v, bufa_v, sema)

        def f1_pair(p, carry):
            c0 = 2 * p
            f1_issue(c0 + 1, idxb_v, bufb_v, semb)
            f1_drain(c0, idxa_v, bufa_v, sema)
            f1_issue(c0 + 2, idxa_v, bufa_v, sema)
            f1_drain(c0 + 1, idxb_v, bufb_v, semb)
            return carry
        lax.fori_loop(0, nchunk1 // 2, f1_pair, 0)

        def x2_issue(c, idx_v, buf_v, sem):
            @pl.when(c < nchunk2)
            def _():
                rbase = (wid * nseg + c * CH) * S2
                pltpu.sync_copy(f2_hbm.at[pl.ds(rbase, ROWS)], idx_v)
                pltpu.async_copy(fm_hbm.at[idx_v], buf_v, sem)

        def x2_drain(c, idx_v, buf_v, sem):
            pltpu.make_async_copy(fm_hbm.at[idx_v], buf_v, sem).wait()

            def seg(s, inner):
                r0 = s * S2
                for kk in range(PW // 16):
                    col = kk * 16
                    w = buf_v[r0, pl.ds(col, 16)]
                    ae = lax.bitcast_convert_type(w << 16, jnp.float32)
                    ao = lax.bitcast_convert_type(
                        w & jnp.int32(-65536), jnp.float32)
                    for j in range(1, S2):
                        w = buf_v[r0 + j, pl.ds(col, 16)]
                        ae = ae + lax.bitcast_convert_type(w << 16, jnp.float32)
                        ao = ao + lax.bitcast_convert_type(
                            w & jnp.int32(-65536), jnp.float32)
                    acc_v[s, pl.ds(kk * 32, 16)] = ae
                    acc_v[s, pl.ds(kk * 32 + 16, 16)] = ao
                return inner
            lax.fori_loop(0, CH, seg, 0)
            pltpu.sync_copy(acc_v, out2.at[pl.ds(wid * nseg + c * CH, CH)])

        x2_issue(0, idxa_v, bufa_v, sema)

        def x2_pair(p, carry):
            c0 = 2 * p
            x2_issue(c0 + 1, idxb_v, bufb_v, semb)
            x2_drain(c0, idxa_v, bufa_v, sema)
            x2_issue(c0 + 2, idxa_v, bufa_v, sema)
            x2_drain(c0 + 1, idxb_v, bufb_v, semb)
            return carry
        lax.fori_loop(0, nchunk2 // 2, x2_pair, 0)

    return sc_kernel(forest0, forest1f, forest2f, fmp)


def _unpack_feat(packed_i32):
    fe = lax.bitcast_convert_type(packed_i32 << 16, jnp.float32)
    fo = lax.bitcast_convert_type(packed_i32 & jnp.int32(-65536), jnp.float32)
    return fe, fo


def _tc_layer1(feat1p, x2s, feat0p, wae, wao, wbe, wbo, wbp):
    N1, PW = feat1p.shape
    F = x2s.shape[1]
    B = feat0p.shape[0]
    R = 2048
    G = R // 16
    grid = N1 // R

    def body(f1_ref, x2_ref, f0_ref, wae_ref, wao_ref, wbe_ref, wbo_ref,
             wbp_ref, h0_ref, h1s_ref):
        fe, fo = _unpack_feat(f1_ref[...])
        h1 = jnp.dot(fe, wae_ref[...], preferred_element_type=jnp.float32)
        h1 = h1 + jnp.dot(fo, wao_ref[...], preferred_element_type=jnp.float32)
        h1 = h1 + jnp.dot(x2_ref[...], wbp_ref[...],
                          preferred_element_type=jnp.float32)
        h1 = jnp.maximum(h1, 0.0)
        h1s_ref[...] = h1.reshape(G, 16, F).sum(axis=1)
        xe = fe.reshape(G, 16, PW).sum(axis=1)
        xo = fo.reshape(G, 16, PW).sum(axis=1)
        f0e, f0o = _unpack_feat(f0_ref[...])
        h0 = jnp.dot(f0e, wae_ref[...], preferred_element_type=jnp.float32)
        h0 = h0 + jnp.dot(f0o, wao_ref[...], preferred_element_type=jnp.float32)
        h0 = h0 + jnp.dot(xe, wbe_ref[...], preferred_element_type=jnp.float32)
        h0 = h0 + jnp.dot(xo, wbo_ref[...], preferred_element_type=jnp.float32)
        h0_ref[...] = jnp.maximum(h0, 0.0)

    return pl.pallas_call(
        body,
        grid=(grid,),
        in_specs=[
            pl.BlockSpec((R, PW), lambda i: (i, 0)),
            pl.BlockSpec((R, F), lambda i: (i, 0)),
            pl.BlockSpec((G, PW), lambda i: (i, 0)),
            pl.BlockSpec((PW, F), lambda i: (0, 0)),
            pl.BlockSpec((PW, F), lambda i: (0, 0)),
            pl.BlockSpec((PW, F), lambda i: (0, 0)),
            pl.BlockSpec((PW, F), lambda i: (0, 0)),
            pl.BlockSpec((F, F), lambda i: (0, 0)),
        ],
        out_specs=[
            pl.BlockSpec((G, F), lambda i: (i, 0)),
            pl.BlockSpec((G, F), lambda i: (i, 0)),
        ],
        out_shape=[
            jax.ShapeDtypeStruct((B, F), jnp.float32),
            jax.ShapeDtypeStruct((B, F), jnp.float32),
        ],
    )(feat1p, x2s, feat0p, wae, wao, wbe, wbo, wbp)


def _tc_layer2(h0, h1s, w2a, w2b16):
    B, H = h0.shape

    def body(h0_ref, h1_ref, wa_ref, wb_ref, out_ref):
        o = jnp.dot(h0_ref[...], wa_ref[...], preferred_element_type=jnp.float32)
        o = o + jnp.dot(h1_ref[...], wb_ref[...], preferred_element_type=jnp.float32)
        out_ref[...] = jnp.maximum(o, 0.0)

    return pl.pallas_call(
        body,
        out_shape=jax.ShapeDtypeStruct((B, H), jnp.float32),
    )(h0, h1s, w2a, w2b16)


def kernel(forest0, forest1, forest2, feature_matrix, W1, W2):
    N, F = feature_matrix.shape
    f0 = forest0.astype(jnp.int32)
    f1 = forest1.reshape(-1).astype(jnp.int32)
    f2 = forest2.reshape(-1).astype(jnp.int32)

    fmp = lax.bitcast_convert_type(
        feature_matrix.astype(jnp.bfloat16).reshape(N, F // 2, 2), jnp.int32)

    feat0p, feat1p, x2s = _sc_gather_all(f0, f1, f2, fmp)

    W1t = W1.T
    w1a = W1t[:F]
    w1b16 = W1t[F:] * (1.0 / 16.0)
    wae, wao = w1a[0::2], w1a[1::2]
    wbe, wbo = w1b16[0::2], w1b16[1::2]
    perm = ((np.arange(F // 32) * 32)[:, None, None]
            + np.array([0, 1])[None, :, None]
            + np.arange(0, 32, 2)[None, None, :]).reshape(F)
    wbp = jnp.take(w1b16, jnp.asarray(perm, dtype=jnp.int32), axis=0)

    W2t = W2.T
    w2a = W2t[:F]
    w2b16 = W2t[F:] * (1.0 / 16.0)

    h0, h1s = _tc_layer1(feat1p, x2s, feat0p, wae, wao, wbe, wbo, wbp)
    return _tc_layer2(h0, h1s, w2a, w2b16)


# speedup vs baseline: 2.8649x; 2.8649x over previous
import functools

import jax
import jax.numpy as jnp
from jax import lax
from jax.experimental import pallas as pl
from jax.experimental.pallas import tpu as pltpu
from jax.experimental.pallas import tpu_sc as plsc

_NC = 2
_NS = 16
_NW = _NC * _NS


def _rne16(u):
    t = (u >> 16) & jnp.int32(0xFFFF)
    r = (u & jnp.int32(0xFFFF)) + jnp.int32(0x7FFF) + ((u >> 16) & jnp.int32(1))
    return (t + ((r >> 16) & jnp.int32(1))) & jnp.int32(0xFFFF)


def _tc_pack(fm):
    N, F = fm.shape
    H = F // 2
    RB = 2000

    def body(x_ref, out_ref):
        u = lax.bitcast_convert_type(x_ref[...], jnp.int32)
        lo = _rne16(u[:, :H])
        hi = _rne16(u[:, H:])
        out_ref[...] = lo | (hi << 16)

    return pl.pallas_call(
        body,
        grid=(N // RB,),
        in_specs=[pl.BlockSpec((RB, F), lambda i: (i, 0))],
        out_specs=pl.BlockSpec((RB, H), lambda i: (i, 0)),
        out_shape=jax.ShapeDtypeStruct((N, H), jnp.int32),
    )(fm)


def _sc_gather_all(forest0, forest1f, forest2f, fmp):
    B = forest0.shape[0]
    N1 = forest1f.shape[0]
    N2 = forest2f.shape[0]
    PW = fmp.shape[1]
    F = 2 * PW
    S2 = N2 // N1
    n0 = B // _NW
    n1 = N1 // _NW
    CH = 8
    ROWS = CH * S2
    nchunk1 = n1 // ROWS
    nseg = (N2 // S2) // _NW
    nchunk2 = nseg // CH

    mesh = plsc.VectorSubcoreMesh(core_axis_name="c", subcore_axis_name="s")

    @functools.partial(
        pl.kernel,
        mesh=mesh,
        out_type=[
            jax.ShapeDtypeStruct((B, PW), jnp.int32),
            jax.ShapeDtypeStruct((N1, PW), jnp.int32),
            jax.ShapeDtypeStruct((N1, F), jnp.float32),
        ],
        scratch_types=[
            pltpu.VMEM((n0,), jnp.int32),
            pltpu.VMEM((ROWS,), jnp.int32),
            pltpu.VMEM((ROWS,), jnp.int32),
            pltpu.VMEM((ROWS, PW), jnp.int32),
            pltpu.VMEM((ROWS, PW), jnp.int32),
            pltpu.VMEM((CH, F), jnp.float32),
            pltpu.SemaphoreType.DMA,
            pltpu.SemaphoreType.DMA,
        ],
    )
    def sc_kernel(f0_hbm, f1_hbm, f2_hbm, fm_hbm, out0, out1, out2,
                  idx0_v, idxa_v, idxb_v, bufa_v, bufb_v, acc_v, sema, semb):
        wid = lax.axis_index("s") * _NC + lax.axis_index("c")

        base0 = wid * n0
        pltpu.sync_copy(f0_hbm.at[pl.ds(base0, n0)], idx0_v)
        pltpu.async_copy(fm_hbm.at[idx0_v], bufa_v.at[pl.ds(0, n0)], sema).wait()
        pltpu.sync_copy(bufa_v.at[pl.ds(0, n0)], out0.at[pl.ds(base0, n0)])

        def f1_issue(c, idx_v, buf_v, sem):
            @pl.when(c < nchunk1)
            def _():
                base = wid * n1 + c * ROWS
                pltpu.sync_copy(f1_hbm.at[pl.ds(base, ROWS)], idx_v)
                pltpu.async_copy(fm_hbm.at[idx_v], buf_v, sem)

        def f1_drain(c, idx_v, buf_v, sem):
            pltpu.make_async_copy(fm_hbm.at[idx_v], buf_v, sem).wait()
            pltpu.sync_copy(buf_v, out1.at[pl.ds(wid * n1 + c * ROWS, ROWS)])

        f1_issue(0, idxa_v, bufa_v, sema)

        def f1_pair(p, carry):
            c0 = 2 * p
            f1_issue(c0 + 1, idxb_v, bufb_v, semb)
            f1_drain(c0, idxa_v, bufa_v, sema)
            f1_issue(c0 + 2, idxa_v, bufa_v, sema)
            f1_drain(c0 + 1, idxb_v, bufb_v, semb)
            return carry
        lax.fori_loop(0, nchunk1 // 2, f1_pair, 0)

        def x2_issue(c, idx_v, buf_v, sem):
            @pl.when(c < nchunk2)
            def _():
                rbase = (wid * nseg + c * CH) * S2
                pltpu.sync_copy(f2_hbm.at[pl.ds(rbase, ROWS)], idx_v)
                pltpu.async_copy(fm_hbm.at[idx_v], buf_v, sem)

        def x2_drain(c, idx_v, buf_v, sem):
            pltpu.make_async_copy(fm_hbm.at[idx_v], buf_v, sem).wait()

            def seg(s, inner):
                r0 = s * S2
                for kk in range(PW // 16):
                    col = kk * 16
                    w = buf_v[r0, pl.ds(col, 16)]
                    alo = lax.bitcast_convert_type(w << 16, jnp.float32)
                    ahi = lax.bitcast_convert_type(
                        w & jnp.int32(-65536), jnp.float32)
                    for j in range(1, S2):
                        w = buf_v[r0 + j, pl.ds(col, 16)]
                        alo = alo + lax.bitcast_convert_type(w << 16, jnp.float32)
                        ahi = ahi + lax.bitcast_convert_type(
                            w & jnp.int32(-65536), jnp.float32)
                    acc_v[s, pl.ds(col, 16)] = alo
                    acc_v[s, pl.ds(PW + col, 16)] = ahi
                return inner
            lax.fori_loop(0, CH, seg, 0)
            pltpu.sync_copy(acc_v, out2.at[pl.ds(wid * nseg + c * CH, CH)])

        x2_issue(0, idxa_v, bufa_v, sema)

        def x2_pair(p, carry):
            c0 = 2 * p
            x2_issue(c0 + 1, idxb_v, bufb_v, semb)
            x2_drain(c0, idxa_v, bufa_v, sema)
            x2_issue(c0 + 2, idxa_v, bufa_v, sema)
            x2_drain(c0 + 1, idxb_v, bufb_v, semb)
            return carry
        lax.fori_loop(0, nchunk2 // 2, x2_pair, 0)

    return sc_kernel(forest0, forest1f, forest2f, fmp)


def _unpack_feat(packed_i32):
    flo = lax.bitcast_convert_type(packed_i32 << 16, jnp.float32)
    fhi = lax.bitcast_convert_type(packed_i32 & jnp.int32(-65536), jnp.float32)
    return flo, fhi


def _tc_layer1(feat1p, x2s, feat0p, walo, wahi, wb16lo, wb16hi):
    N1, PW = feat1p.shape
    F = x2s.shape[1]
    B = feat0p.shape[0]
    R = 2048
    G = R // 16
    grid = N1 // R

    def body(f1_ref, x2_ref, f0_ref, walo_ref, wahi_ref, wblo_ref, wbhi_ref,
             h0_ref, h1s_ref):
        flo, fhi = _unpack_feat(f1_ref[...])
        x2 = x2_ref[...]
        h1 = jnp.dot(flo, walo_ref[...], preferred_element_type=jnp.float32)
        h1 = h1 + jnp.dot(fhi, wahi_ref[...], preferred_element_type=jnp.float32)
        h1 = h1 + jnp.dot(x2[:, :PW], wblo_ref[...],
                          preferred_element_type=jnp.float32)
        h1 = h1 + jnp.dot(x2[:, PW:], wbhi_ref[...],
                          preferred_element_type=jnp.float32)
        h1 = jnp.maximum(h1, 0.0)
        h1s_ref[...] = h1.reshape(G, 16, F).sum(axis=1)
        xlo = flo.reshape(G, 16, PW).sum(axis=1)
        xhi = fhi.reshape(G, 16, PW).sum(axis=1)
        f0lo, f0hi = _unpack_feat(f0_ref[...])
        h0 = jnp.dot(f0lo, walo_ref[...], preferred_element_type=jnp.float32)
        h0 = h0 + jnp.dot(f0hi, wahi_ref[...], preferred_element_type=jnp.float32)
        h0 = h0 + jnp.dot(xlo, wblo_ref[...], preferred_element_type=jnp.float32)
        h0 = h0 + jnp.dot(xhi, wbhi_ref[...], preferred_element_type=jnp.float32)
        h0_ref[...] = jnp.maximum(h0, 0.0)

    return pl.pallas_call(
        body,
        grid=(grid,),
        in_specs=[
            pl.BlockSpec((R, PW), lambda i: (i, 0)),
            pl.BlockSpec((R, F), lambda i: (i, 0)),
            pl.BlockSpec((G, PW), lambda i: (i, 0)),
            pl.BlockSpec((PW, F), lambda i: (0, 0)),
            pl.BlockSpec((PW, F), lambda i: (0, 0)),
            pl.BlockSpec((PW, F), lambda i: (0, 0)),
            pl.BlockSpec((PW, F), lambda i: (0, 0)),
        ],
        out_specs=[
            pl.BlockSpec((G, F), lambda i: (i, 0)),
            pl.BlockSpec((G, F), lambda i: (i, 0)),
        ],
        out_shape=[
            jax.ShapeDtypeStruct((B, F), jnp.float32),
            jax.ShapeDtypeStruct((B, F), jnp.float32),
        ],
    )(feat1p, x2s, feat0p, walo, wahi, wb16lo, wb16hi)


def _tc_layer2(h0, h1s, w2a, w2b16):
    B, H = h0.shape

    def body(h0_ref, h1_ref, wa_ref, wb_ref, out_ref):
        o = jnp.dot(h0_ref[...], wa_ref[...], preferred_element_type=jnp.float32)
        o = o + jnp.dot(h1_ref[...], wb_ref[...], preferred_element_type=jnp.float32)
        out_ref[...] = jnp.maximum(o, 0.0)

    return pl.pallas_call(
        body,
        out_shape=jax.ShapeDtypeStruct((B, H), jnp.float32),
    )(h0, h1s, w2a, w2b16)


def kernel(forest0, forest1, forest2, feature_matrix, W1, W2):
    N, F = feature_matrix.shape
    H = F // 2
    f0 = forest0.astype(jnp.int32)
    f1 = forest1.reshape(-1).astype(jnp.int32)
    f2 = forest2.reshape(-1).astype(jnp.int32)

    fmp = _tc_pack(feature_matrix)

    feat0p, feat1p, x2s = _sc_gather_all(f0, f1, f2, fmp)

    W1t = W1.T
    w1a = W1t[:F]
    w1b16 = W1t[F:] * (1.0 / 16.0)
    walo, wahi = w1a[:H], w1a[H:]
    wb16lo, wb16hi = w1b16[:H], w1b16[H:]

    W2t = W2.T
    w2a = W2t[:F]
    w2b16 = W2t[F:] * (1.0 / 16.0)

    h0, h1s = _tc_layer1(feat1p, x2s, feat0p, walo, wahi, wb16lo, wb16hi)
    return _tc_layer2(h0, h1s, w2a, w2b16)


# speedup vs baseline: 3.3026x; 1.1528x over previous
import functools

import jax
import jax.numpy as jnp
from jax import lax
from jax.experimental import pallas as pl
from jax.experimental.pallas import tpu as pltpu
from jax.experimental.pallas import tpu_sc as plsc

_NC = 2
_NS = 16
_NW = _NC * _NS


def _rne16(u):
    t = (u >> 16) & jnp.int32(0xFFFF)
    r = (u & jnp.int32(0xFFFF)) + jnp.int32(0x7FFF) + ((u >> 16) & jnp.int32(1))
    return (t + ((r >> 16) & jnp.int32(1))) & jnp.int32(0xFFFF)


def _tc_pack(fm):
    N, F = fm.shape
    H = F // 2
    RB = 2000

    def body(x_ref, out_ref):
        u = lax.bitcast_convert_type(x_ref[...], jnp.int32)
        lo = _rne16(u[:, :H])
        hi = _rne16(u[:, H:])
        out_ref[...] = lo | (hi << 16)

    return pl.pallas_call(
        body,
        grid=(N // RB,),
        in_specs=[pl.BlockSpec((RB, F), lambda i: (i, 0))],
        out_specs=pl.BlockSpec((RB, H), lambda i: (i, 0)),
        out_shape=jax.ShapeDtypeStruct((N, H), jnp.int32),
    )(fm)


def _sc_gather_all(forest0, forest1f, forest2f, fmp):
    B = forest0.shape[0]
    N1 = forest1f.size
    N2 = forest2f.size
    PW = fmp.shape[1]
    F = 2 * PW
    S2 = N2 // N1
    n0 = B // _NW
    n1 = N1 // _NW
    CH = 8
    ROWS = CH * S2
    nchunk1 = n1 // ROWS
    nseg = (N2 // S2) // _NW
    nchunk2 = nseg // CH

    mesh = plsc.VectorSubcoreMesh(core_axis_name="c", subcore_axis_name="s")

    @functools.partial(
        pl.kernel,
        mesh=mesh,
        out_type=[
            jax.ShapeDtypeStruct((B, PW), jnp.int32),
            jax.ShapeDtypeStruct((N1, PW), jnp.int32),
            jax.ShapeDtypeStruct((N1, F), jnp.float32),
        ],
        scratch_types=[
            pltpu.VMEM((n0,), jnp.int32),
            pltpu.VMEM((nchunk1, ROWS), jnp.int32),
            pltpu.VMEM((nchunk2, ROWS), jnp.int32),
            pltpu.VMEM((ROWS, PW), jnp.int32),
            pltpu.VMEM((ROWS, PW), jnp.int32),
            pltpu.VMEM((CH, F), jnp.float32),
            pltpu.SemaphoreType.DMA,
            pltpu.SemaphoreType.DMA,
        ],
    )
    def sc_kernel(f0_hbm, f1_hbm, f2_hbm, fm_hbm, out0, out1, out2,
                  idx0_v, idx1_v, idx2_v, bufa_v, bufb_v, acc_v, sema, semb):
        wid = lax.axis_index("s") * _NC + lax.axis_index("c")

        pltpu.sync_copy(f1_hbm.at[pl.ds(wid * nchunk1, nchunk1)], idx1_v)
        pltpu.sync_copy(f2_hbm.at[pl.ds(wid * nchunk2, nchunk2)], idx2_v)

        base0 = wid * n0
        pltpu.sync_copy(f0_hbm.at[pl.ds(base0, n0)], idx0_v)
        pltpu.async_copy(fm_hbm.at[idx0_v], bufa_v.at[pl.ds(0, n0)], sema).wait()
        pltpu.sync_copy(bufa_v.at[pl.ds(0, n0)], out0.at[pl.ds(base0, n0)])

        def f1_issue(c, buf_v, sem):
            @pl.when(c < nchunk1)
            def _():
                pltpu.async_copy(fm_hbm.at[idx1_v.at[c]], buf_v, sem)

        def f1_drain(c, buf_v, sem):
            pltpu.make_async_copy(fm_hbm.at[idx1_v.at[c]], buf_v, sem).wait()
            pltpu.sync_copy(buf_v, out1.at[pl.ds(wid * n1 + c * ROWS, ROWS)])

        f1_issue(0, bufa_v, sema)

        def f1_pair(p, carry):
            c0 = 2 * p
            f1_issue(c0 + 1, bufb_v, semb)
            f1_drain(c0, bufa_v, sema)
            f1_issue(c0 + 2, bufa_v, sema)
            f1_drain(c0 + 1, bufb_v, semb)
            return carry
        lax.fori_loop(0, nchunk1 // 2, f1_pair, 0)

        def x2_issue(c, buf_v, sem):
            @pl.when(c < nchunk2)
            def _():
                pltpu.async_copy(fm_hbm.at[idx2_v.at[c]], buf_v, sem)

        def x2_drain(c, buf_v, sem):
            pltpu.make_async_copy(fm_hbm.at[idx2_v.at[c]], buf_v, sem).wait()

            def seg(s, inner):
                r0 = s * S2
                for kk in range(PW // 16):
                    col = kk * 16
                    w = buf_v[r0, pl.ds(col, 16)]
                    alo = lax.bitcast_convert_type(w << 16, jnp.float32)
                    ahi = lax.bitcast_convert_type(
                        w & jnp.int32(-65536), jnp.float32)
                    for j in range(1, S2):
                        w = buf_v[r0 + j, pl.ds(col, 16)]
                        alo = alo + lax.bitcast_convert_type(w << 16, jnp.float32)
                        ahi = ahi + lax.bitcast_convert_type(
                            w & jnp.int32(-65536), jnp.float32)
                    acc_v[s, pl.ds(col, 16)] = alo
                    acc_v[s, pl.ds(PW + col, 16)] = ahi
                return inner
            lax.fori_loop(0, CH, seg, 0)
            pltpu.sync_copy(acc_v, out2.at[pl.ds(wid * nseg + c * CH, CH)])

        x2_issue(0, bufa_v, sema)

        def x2_pair(p, carry):
            c0 = 2 * p
            x2_issue(c0 + 1, bufb_v, semb)
            x2_drain(c0, bufa_v, sema)
            x2_issue(c0 + 2, bufa_v, sema)
            x2_drain(c0 + 1, bufb_v, semb)
            return carry
        lax.fori_loop(0, nchunk2 // 2, x2_pair, 0)

    return sc_kernel(forest0, forest1f, forest2f, fmp)


def _unpack_feat(packed_i32):
    flo = lax.bitcast_convert_type(packed_i32 << 16, jnp.float32)
    fhi = lax.bitcast_convert_type(packed_i32 & jnp.int32(-65536), jnp.float32)
    return flo, fhi


def _tc_layer1(feat1p, x2s, feat0p, walo, wahi, wb16lo, wb16hi):
    N1, PW = feat1p.shape
    F = x2s.shape[1]
    B = feat0p.shape[0]
    R = 2048
    G = R // 16
    grid = N1 // R

    def body(f1_ref, x2_ref, f0_ref, walo_ref, wahi_ref, wblo_ref, wbhi_ref,
             h0_ref, h1s_ref):
        flo, fhi = _unpack_feat(f1_ref[...])
        x2 = x2_ref[...]
        h1 = jnp.dot(flo, walo_ref[...], preferred_element_type=jnp.float32)
        h1 = h1 + jnp.dot(fhi, wahi_ref[...], preferred_element_type=jnp.float32)
        h1 = h1 + jnp.dot(x2[:, :PW], wblo_ref[...],
                          preferred_element_type=jnp.float32)
        h1 = h1 + jnp.dot(x2[:, PW:], wbhi_ref[...],
                          preferred_element_type=jnp.float32)
        h1 = jnp.maximum(h1, 0.0)
        h1s_ref[...] = h1.reshape(G, 16, F).sum(axis=1)
        xlo = flo.reshape(G, 16, PW).sum(axis=1)
        xhi = fhi.reshape(G, 16, PW).sum(axis=1)
        f0lo, f0hi = _unpack_feat(f0_ref[...])
        h0 = jnp.dot(f0lo, walo_ref[...], preferred_element_type=jnp.float32)
        h0 = h0 + jnp.dot(f0hi, wahi_ref[...], preferred_element_type=jnp.float32)
        h0 = h0 + jnp.dot(xlo, wblo_ref[...], preferred_element_type=jnp.float32)
        h0 = h0 + jnp.dot(xhi, wbhi_ref[...], preferred_element_type=jnp.float32)
        h0_ref[...] = jnp.maximum(h0, 0.0)

    return pl.pallas_call(
        body,
        grid=(grid,),
        in_specs=[
            pl.BlockSpec((R, PW), lambda i: (i, 0)),
            pl.BlockSpec((R, F), lambda i: (i, 0)),
            pl.BlockSpec((G, PW), lambda i: (i, 0)),
            pl.BlockSpec((PW, F), lambda i: (0, 0)),
            pl.BlockSpec((PW, F), lambda i: (0, 0)),
            pl.BlockSpec((PW, F), lambda i: (0, 0)),
            pl.BlockSpec((PW, F), lambda i: (0, 0)),
        ],
        out_specs=[
            pl.BlockSpec((G, F), lambda i: (i, 0)),
            pl.BlockSpec((G, F), lambda i: (i, 0)),
        ],
        out_shape=[
            jax.ShapeDtypeStruct((B, F), jnp.float32),
            jax.ShapeDtypeStruct((B, F), jnp.float32),
        ],
    )(feat1p, x2s, feat0p, walo, wahi, wb16lo, wb16hi)


def _tc_layer2(h0, h1s, w2a, w2b16):
    B, H = h0.shape

    def body(h0_ref, h1_ref, wa_ref, wb_ref, out_ref):
        o = jnp.dot(h0_ref[...], wa_ref[...], preferred_element_type=jnp.float32)
        o = o + jnp.dot(h1_ref[...], wb_ref[...], preferred_element_type=jnp.float32)
        out_ref[...] = jnp.maximum(o, 0.0)

    return pl.pallas_call(
        body,
        out_shape=jax.ShapeDtypeStruct((B, H), jnp.float32),
    )(h0, h1s, w2a, w2b16)


def kernel(forest0, forest1, forest2, feature_matrix, W1, W2):
    N, F = feature_matrix.shape
    H = F // 2
    f0 = forest0.astype(jnp.int32)
    f1 = forest1.reshape(-1).astype(jnp.int32)
    f2 = forest2.reshape(-1).astype(jnp.int32)

    fmp = _tc_pack(feature_matrix)

    feat0p, feat1p, x2s = _sc_gather_all(f0, f1.reshape(-1, 128), f2.reshape(-1, 128), fmp)

    W1t = W1.T
    w1a = W1t[:F]
    w1b16 = W1t[F:] * (1.0 / 16.0)
    walo, wahi = w1a[:H], w1a[H:]
    wb16lo, wb16hi = w1b16[:H], w1b16[H:]

    W2t = W2.T
    w2a = W2t[:F]
    w2b16 = W2t[F:] * (1.0 / 16.0)

    h0, h1s = _tc_layer1(feat1p, x2s, feat0p, walo, wahi, wb16lo, wb16hi)
    return _tc_layer2(h0, h1s, w2a, w2b16)


# speedup vs baseline: 3.3063x; 1.0011x over previous
import functools

import jax
import jax.numpy as jnp
from jax import lax
from jax.experimental import pallas as pl
from jax.experimental.pallas import tpu as pltpu
from jax.experimental.pallas import tpu_sc as plsc

_NC = 2
_NS = 16
_NW = _NC * _NS


def _rne16(u):
    t = (u >> 16) & jnp.int32(0xFFFF)
    r = (u & jnp.int32(0xFFFF)) + jnp.int32(0x7FFF) + ((u >> 16) & jnp.int32(1))
    return (t + ((r >> 16) & jnp.int32(1))) & jnp.int32(0xFFFF)


def _tc_pack(fm):
    N, F = fm.shape
    H = F // 2
    RB = 2000

    def body(x_ref, out_ref):
        u = lax.bitcast_convert_type(x_ref[...], jnp.int32)
        lo = _rne16(u[:, :H])
        hi = _rne16(u[:, H:])
        out_ref[...] = lo | (hi << 16)

    return pl.pallas_call(
        body,
        grid=(N // RB,),
        in_specs=[pl.BlockSpec((RB, F), lambda i: (i, 0))],
        out_specs=pl.BlockSpec((RB, H), lambda i: (i, 0)),
        out_shape=jax.ShapeDtypeStruct((N, H), jnp.int32),
    )(fm)


def _sc_gather_all(forest0, forest1f, forest2f, fmp):
    B = forest0.shape[0]
    N1 = forest1f.size
    N2 = forest2f.size
    PW = fmp.shape[1]
    F = 2 * PW
    S2 = N2 // N1
    n0 = B // _NW
    n1 = N1 // _NW
    CH = 8
    ROWS = CH * S2
    nchunk1 = n1 // ROWS
    nseg = (N2 // S2) // _NW
    nchunk2 = nseg // CH

    mesh = plsc.VectorSubcoreMesh(core_axis_name="c", subcore_axis_name="s")

    @functools.partial(
        pl.kernel,
        mesh=mesh,
        out_type=[
            jax.ShapeDtypeStruct((B, PW), jnp.int32),
            jax.ShapeDtypeStruct((N1, PW), jnp.int32),
            jax.ShapeDtypeStruct((N1, F), jnp.float32),
        ],
        scratch_types=[
            pltpu.VMEM((n0,), jnp.int32),
            pltpu.VMEM((nchunk1, ROWS), jnp.int32),
            pltpu.VMEM((nchunk2, ROWS), jnp.int32),
            pltpu.VMEM((ROWS, PW), jnp.int32),
            pltpu.VMEM((ROWS, PW), jnp.int32),
            pltpu.VMEM((ROWS, PW), jnp.int32),
            pltpu.VMEM((ROWS, PW), jnp.int32),
            pltpu.VMEM((ROWS, PW), jnp.int32),
            pltpu.VMEM((ROWS, PW), jnp.int32),
            pltpu.VMEM((CH, F), jnp.float32),
            pltpu.SemaphoreType.DMA,
            pltpu.SemaphoreType.DMA,
            pltpu.SemaphoreType.DMA,
            pltpu.SemaphoreType.DMA,
            pltpu.SemaphoreType.DMA,
            pltpu.SemaphoreType.DMA,
        ],
    )
    def sc_kernel(f0_hbm, f1_hbm, f2_hbm, fm_hbm, out0, out1, out2,
                  idx0_v, idx1_v, idx2_v, bufa_v, bufb_v, bufc_v, bufd_v,
                  bufe_v, buff_v, acc_v, sema, semb, semc, semd, seme, semf):
        wid = lax.axis_index("s") * _NC + lax.axis_index("c")

        pltpu.sync_copy(f1_hbm.at[pl.ds(wid * nchunk1, nchunk1)], idx1_v)
        pltpu.sync_copy(f2_hbm.at[pl.ds(wid * nchunk2, nchunk2)], idx2_v)

        def x2_issue(c, buf_v, sem):
            @pl.when(c < nchunk2)
            def _():
                pltpu.async_copy(fm_hbm.at[idx2_v.at[c]], buf_v, sem)

        x2_issue(0, bufa_v, sema)
        x2_issue(1, bufb_v, semb)
        x2_issue(2, bufc_v, semc)

        base0 = wid * n0
        pltpu.sync_copy(f0_hbm.at[pl.ds(base0, n0)], idx0_v)
        pltpu.async_copy(fm_hbm.at[idx0_v], bufe_v.at[pl.ds(0, n0)], seme).wait()
        pltpu.sync_copy(bufe_v.at[pl.ds(0, n0)], out0.at[pl.ds(base0, n0)])

        def f1_issue(c, buf_v, sem):
            @pl.when(c < nchunk1)
            def _():
                pltpu.async_copy(fm_hbm.at[idx1_v.at[c]], buf_v, sem)

        def f1_drain(c, buf_v, sem):
            pltpu.make_async_copy(fm_hbm.at[idx1_v.at[c]], buf_v, sem).wait()
            pltpu.sync_copy(buf_v, out1.at[pl.ds(wid * n1 + c * ROWS, ROWS)])

        f1_issue(0, bufe_v, seme)

        def f1_pair(p, carry):
            c0 = 2 * p
            f1_issue(c0 + 1, buff_v, semf)
            f1_drain(c0, bufe_v, seme)
            f1_issue(c0 + 2, bufe_v, seme)
            f1_drain(c0 + 1, buff_v, semf)
            return carry
        lax.fori_loop(0, nchunk1 // 2, f1_pair, 0)


        def x2_drain(c, buf_v, sem):
            pltpu.make_async_copy(fm_hbm.at[idx2_v.at[c]], buf_v, sem).wait()

            def seg(s, inner):
                r0 = s * S2
                for kk in range(PW // 16):
                    col = kk * 16
                    w = buf_v[r0, pl.ds(col, 16)]
                    alo = lax.bitcast_convert_type(w << 16, jnp.float32)
                    ahi = lax.bitcast_convert_type(
                        w & jnp.int32(-65536), jnp.float32)
                    for j in range(1, S2):
                        w = buf_v[r0 + j, pl.ds(col, 16)]
                        alo = alo + lax.bitcast_convert_type(w << 16, jnp.float32)
                        ahi = ahi + lax.bitcast_convert_type(
                            w & jnp.int32(-65536), jnp.float32)
                    acc_v[s, pl.ds(col, 16)] = alo
                    acc_v[s, pl.ds(PW + col, 16)] = ahi
                return inner
            lax.fori_loop(0, CH, seg, 0)
            pltpu.sync_copy(acc_v, out2.at[pl.ds(wid * nseg + c * CH, CH)])

        def x2_quad(p, carry):
            c0 = 4 * p
            x2_issue(c0 + 3, bufd_v, semd)
            x2_drain(c0, bufa_v, sema)
            x2_issue(c0 + 4, bufa_v, sema)
            x2_drain(c0 + 1, bufb_v, semb)
            x2_issue(c0 + 5, bufb_v, semb)
            x2_drain(c0 + 2, bufc_v, semc)
            x2_issue(c0 + 6, bufc_v, semc)
            x2_drain(c0 + 3, bufd_v, semd)
            return carry
        lax.fori_loop(0, nchunk2 // 4, x2_quad, 0)

    return sc_kernel(forest0, forest1f, forest2f, fmp)


def _unpack_feat(packed_i32):
    flo = lax.bitcast_convert_type(packed_i32 << 16, jnp.float32)
    fhi = lax.bitcast_convert_type(packed_i32 & jnp.int32(-65536), jnp.float32)
    return flo, fhi


def _tc_layer1(feat1p, x2s, feat0p, walo, wahi, wb16lo, wb16hi):
    N1, PW = feat1p.shape
    F = x2s.shape[1]
    B = feat0p.shape[0]
    R = 2048
    G = R // 16
    grid = N1 // R

    def body(f1_ref, x2_ref, f0_ref, walo_ref, wahi_ref, wblo_ref, wbhi_ref,
             h0_ref, h1s_ref):
        flo, fhi = _unpack_feat(f1_ref[...])
        x2 = x2_ref[...]
        h1 = jnp.dot(flo, walo_ref[...], preferred_element_type=jnp.float32)
        h1 = h1 + jnp.dot(fhi, wahi_ref[...], preferred_element_type=jnp.float32)
        h1 = h1 + jnp.dot(x2[:, :PW], wblo_ref[...],
                          preferred_element_type=jnp.float32)
        h1 = h1 + jnp.dot(x2[:, PW:], wbhi_ref[...],
                          preferred_element_type=jnp.float32)
        h1 = jnp.maximum(h1, 0.0)
        h1s_ref[...] = h1.reshape(G, 16, F).sum(axis=1)
        xlo = flo.reshape(G, 16, PW).sum(axis=1)
        xhi = fhi.reshape(G, 16, PW).sum(axis=1)
        f0lo, f0hi = _unpack_feat(f0_ref[...])
        h0 = jnp.dot(f0lo, walo_ref[...], preferred_element_type=jnp.float32)
        h0 = h0 + jnp.dot(f0hi, wahi_ref[...], preferred_element_type=jnp.float32)
        h0 = h0 + jnp.dot(xlo, wblo_ref[...], preferred_element_type=jnp.float32)
        h0 = h0 + jnp.dot(xhi, wbhi_ref[...], preferred_element_type=jnp.float32)
        h0_ref[...] = jnp.maximum(h0, 0.0)

    return pl.pallas_call(
        body,
        grid=(grid,),
        in_specs=[
            pl.BlockSpec((R, PW), lambda i: (i, 0)),
            pl.BlockSpec((R, F), lambda i: (i, 0)),
            pl.BlockSpec((G, PW), lambda i: (i, 0)),
            pl.BlockSpec((PW, F), lambda i: (0, 0)),
            pl.BlockSpec((PW, F), lambda i: (0, 0)),
            pl.BlockSpec((PW, F), lambda i: (0, 0)),
            pl.BlockSpec((PW, F), lambda i: (0, 0)),
        ],
        out_specs=[
            pl.BlockSpec((G, F), lambda i: (i, 0)),
            pl.BlockSpec((G, F), lambda i: (i, 0)),
        ],
        out_shape=[
            jax.ShapeDtypeStruct((B, F), jnp.float32),
            jax.ShapeDtypeStruct((B, F), jnp.float32),
        ],
    )(feat1p, x2s, feat0p, walo, wahi, wb16lo, wb16hi)


def _tc_layer2(h0, h1s, w2a, w2b16):
    B, H = h0.shape

    def body(h0_ref, h1_ref, wa_ref, wb_ref, out_ref):
        o = jnp.dot(h0_ref[...], wa_ref[...], preferred_element_type=jnp.float32)
        o = o + jnp.dot(h1_ref[...], wb_ref[...], preferred_element_type=jnp.float32)
        out_ref[...] = jnp.maximum(o, 0.0)

    return pl.pallas_call(
        body,
        out_shape=jax.ShapeDtypeStruct((B, H), jnp.float32),
    )(h0, h1s, w2a, w2b16)


def kernel(forest0, forest1, forest2, feature_matrix, W1, W2):
    N, F = feature_matrix.shape
    H = F // 2
    f0 = forest0.astype(jnp.int32)
    f1 = forest1.reshape(-1).astype(jnp.int32)
    f2 = forest2.reshape(-1).astype(jnp.int32)

    fmp = _tc_pack(feature_matrix)

    feat0p, feat1p, x2s = _sc_gather_all(f0, f1.reshape(-1, 128), f2.reshape(-1, 128), fmp)

    W1t = W1.T
    w1a = W1t[:F]
    w1b16 = W1t[F:] * (1.0 / 16.0)
    walo, wahi = w1a[:H], w1a[H:]
    wb16lo, wb16hi = w1b16[:H], w1b16[H:]

    W2t = W2.T
    w2a = W2t[:F]
    w2b16 = W2t[F:] * (1.0 / 16.0)

    h0, h1s = _tc_layer1(feat1p, x2s, feat0p, walo, wahi, wb16lo, wb16hi)
    return _tc_layer2(h0, h1s, w2a, w2b16)


# speedup vs baseline: 3.4051x; 1.0299x over previous
import functools

import jax
import jax.numpy as jnp
from jax import lax
from jax.experimental import pallas as pl
from jax.experimental.pallas import tpu as pltpu
from jax.experimental.pallas import tpu_sc as plsc

_NC = 2
_NS = 16
_NW = _NC * _NS


def _rne16(u):
    t = (u >> 16) & jnp.int32(0xFFFF)
    r = (u & jnp.int32(0xFFFF)) + jnp.int32(0x7FFF) + ((u >> 16) & jnp.int32(1))
    return (t + ((r >> 16) & jnp.int32(1))) & jnp.int32(0xFFFF)


def _tc_pack(fm):
    N, F = fm.shape
    H = F // 2
    RB = 2000

    def body(x_ref, out_ref):
        u = lax.bitcast_convert_type(x_ref[...], jnp.int32)
        lo = _rne16(u[:, :H])
        hi = _rne16(u[:, H:])
        out_ref[...] = lo | (hi << 16)

    return pl.pallas_call(
        body,
        grid=(N // RB,),
        in_specs=[pl.BlockSpec((RB, F), lambda i: (i, 0))],
        out_specs=pl.BlockSpec((RB, H), lambda i: (i, 0)),
        out_shape=jax.ShapeDtypeStruct((N, H), jnp.int32),
    )(fm)


def _sc_gather_all(forest0, forest1f, forest2f, fmp):
    B = forest0.shape[0]
    N1 = forest1f.size
    N2 = forest2f.size
    PW = fmp.shape[1]
    F = 2 * PW
    S2 = N2 // N1
    n0 = B // _NW
    n1 = N1 // _NW
    CH = 8
    ROWS = CH * S2
    nchunk1 = n1 // ROWS
    nseg = (N2 // S2) // _NW
    nchunk2 = nseg // CH

    mesh = plsc.VectorSubcoreMesh(core_axis_name="c", subcore_axis_name="s")

    @functools.partial(
        pl.kernel,
        mesh=mesh,
        out_type=[
            jax.ShapeDtypeStruct((B, PW), jnp.int32),
            jax.ShapeDtypeStruct((N1, PW), jnp.int32),
            jax.ShapeDtypeStruct((N1, F), jnp.float32),
        ],
        scratch_types=[
            pltpu.VMEM((n0,), jnp.int32),
            pltpu.VMEM((nchunk1, ROWS), jnp.int32),
            pltpu.VMEM((nchunk2, ROWS), jnp.int32),
            pltpu.VMEM((ROWS, PW), jnp.int32),
            pltpu.VMEM((ROWS, PW), jnp.int32),
            pltpu.VMEM((ROWS, PW), jnp.int32),
            pltpu.VMEM((ROWS, PW), jnp.int32),
            pltpu.VMEM((ROWS, PW), jnp.int32),
            pltpu.VMEM((ROWS, PW), jnp.int32),
            pltpu.VMEM((CH, F), jnp.float32),
            pltpu.VMEM((CH, F), jnp.float32),
            pltpu.VMEM((CH, F), jnp.float32),
            pltpu.VMEM((CH, F), jnp.float32),
            pltpu.SemaphoreType.DMA,
            pltpu.SemaphoreType.DMA,
            pltpu.SemaphoreType.DMA,
            pltpu.SemaphoreType.DMA,
            pltpu.SemaphoreType.DMA,
            pltpu.SemaphoreType.DMA,
            pltpu.SemaphoreType.DMA,
            pltpu.SemaphoreType.DMA,
            pltpu.SemaphoreType.DMA,
            pltpu.SemaphoreType.DMA,
            pltpu.SemaphoreType.DMA,
            pltpu.SemaphoreType.DMA,
        ],
    )
    def sc_kernel(f0_hbm, f1_hbm, f2_hbm, fm_hbm, out0, out1, out2,
                  idx0_v, idx1_v, idx2_v, bufa_v, bufb_v, bufc_v, bufd_v,
                  bufe_v, buff_v, acca_v, accb_v, accc_v, accd_v,
                  sema, semb, semc, semd, seme, semf,
                  semwa, semwb, semwc, semwd, semwe, semwf):
        wid = lax.axis_index("s") * _NC + lax.axis_index("c")

        pltpu.sync_copy(f1_hbm.at[pl.ds(wid * nchunk1, nchunk1)], idx1_v)
        pltpu.sync_copy(f2_hbm.at[pl.ds(wid * nchunk2, nchunk2)], idx2_v)

        def x2_issue(c, buf_v, sem):
            @pl.when(c < nchunk2)
            def _():
                pltpu.async_copy(fm_hbm.at[idx2_v.at[c]], buf_v, sem)

        x2_issue(0, bufa_v, sema)
        x2_issue(1, bufb_v, semb)
        x2_issue(2, bufc_v, semc)

        base0 = wid * n0
        pltpu.sync_copy(f0_hbm.at[pl.ds(base0, n0)], idx0_v)
        pltpu.async_copy(fm_hbm.at[idx0_v], bufe_v.at[pl.ds(0, n0)], seme).wait()
        pltpu.sync_copy(bufe_v.at[pl.ds(0, n0)], out0.at[pl.ds(base0, n0)])

        def f1_issue(c, buf_v, sem):
            @pl.when(c < nchunk1)
            def _():
                pltpu.async_copy(fm_hbm.at[idx1_v.at[c]], buf_v, sem)

        def f1_wait_write(buf_v, semw):
            pltpu.make_async_copy(
                buf_v, out1.at[pl.ds(wid * n1, ROWS)], semw).wait()

        def f1_drain(c, buf_v, sem, semw, first):
            pltpu.make_async_copy(fm_hbm.at[idx1_v.at[c]], buf_v, sem).wait()
            pltpu.async_copy(buf_v, out1.at[pl.ds(wid * n1 + c * ROWS, ROWS)],
                             semw)

        f1_issue(0, bufe_v, seme)
        f1_issue(1, buff_v, semf)
        f1_drain(0, bufe_v, seme, semwe, True)
        f1_drain(1, buff_v, semf, semwf, True)
        for c in range(2, nchunk1):
            buf_v = bufe_v if c % 2 == 0 else buff_v
            sem = seme if c % 2 == 0 else semf
            semw = semwe if c % 2 == 0 else semwf
            f1_wait_write(buf_v, semw)
            f1_issue(c, buf_v, sem)
            f1_drain(c, buf_v, sem, semw, False)
        f1_wait_write(bufe_v, semwe)
        f1_wait_write(buff_v, semwf)


        def x2_wait_write(acc_v, semw):
            pltpu.make_async_copy(
                acc_v, out2.at[pl.ds(wid * nseg, CH)], semw).wait()

        def x2_drain(c, buf_v, sem, acc_v, semw, wait_prev):
            pltpu.make_async_copy(fm_hbm.at[idx2_v.at[c]], buf_v, sem).wait()
            if wait_prev:
                x2_wait_write(acc_v, semw)

            def seg(s, inner):
                r0 = s * S2
                for kk in range(PW // 16):
                    col = kk * 16
                    w = buf_v[r0, pl.ds(col, 16)]
                    alo = lax.bitcast_convert_type(w << 16, jnp.float32)
                    ahi = lax.bitcast_convert_type(
                        w & jnp.int32(-65536), jnp.float32)
                    for j in range(1, S2):
                        w = buf_v[r0 + j, pl.ds(col, 16)]
                        alo = alo + lax.bitcast_convert_type(w << 16, jnp.float32)
                        ahi = ahi + lax.bitcast_convert_type(
                            w & jnp.int32(-65536), jnp.float32)
                    acc_v[s, pl.ds(col, 16)] = alo
                    acc_v[s, pl.ds(PW + col, 16)] = ahi
                return inner
            lax.fori_loop(0, CH, seg, 0)
            pltpu.async_copy(acc_v, out2.at[pl.ds(wid * nseg + c * CH, CH)],
                             semw)

        def x2_quad_body(p, c0, wait_prev):
            x2_issue(c0 + 3, bufd_v, semd)
            x2_drain(c0, bufa_v, sema, acca_v, semwa, wait_prev)
            x2_issue(c0 + 4, bufa_v, sema)
            x2_drain(c0 + 1, bufb_v, semb, accb_v, semwb, wait_prev)
            x2_issue(c0 + 5, bufb_v, semb)
            x2_drain(c0 + 2, bufc_v, semc, accc_v, semwc, wait_prev)
            x2_issue(c0 + 6, bufc_v, semc)
            x2_drain(c0 + 3, bufd_v, semd, accd_v, semwd, wait_prev)

        x2_quad_body(0, 0, False)

        def x2_quad(p, carry):
            x2_quad_body(p, 4 * p, True)
            return carry
        lax.fori_loop(1, nchunk2 // 4, x2_quad, 0)

        x2_wait_write(acca_v, semwa)
        x2_wait_write(accb_v, semwb)
        x2_wait_write(accc_v, semwc)
        x2_wait_write(accd_v, semwd)

    return sc_kernel(forest0, forest1f, forest2f, fmp)


def _unpack_feat(packed_i32):
    flo = lax.bitcast_convert_type(packed_i32 << 16, jnp.float32)
    fhi = lax.bitcast_convert_type(packed_i32 & jnp.int32(-65536), jnp.float32)
    return flo, fhi


def _tc_layer1(feat1p, x2s, feat0p, walo, wahi, wb16lo, wb16hi):
    N1, PW = feat1p.shape
    F = x2s.shape[1]
    B = feat0p.shape[0]
    R = 2048
    G = R // 16
    grid = N1 // R

    def body(f1_ref, x2_ref, f0_ref, walo_ref, wahi_ref, wblo_ref, wbhi_ref,
             h0_ref, h1s_ref):
        flo, fhi = _unpack_feat(f1_ref[...])
        x2 = x2_ref[...]
        h1 = jnp.dot(flo, walo_ref[...], preferred_element_type=jnp.float32)
        h1 = h1 + jnp.dot(fhi, wahi_ref[...], preferred_element_type=jnp.float32)
        h1 = h1 + jnp.dot(x2[:, :PW], wblo_ref[...],
                          preferred_element_type=jnp.float32)
        h1 = h1 + jnp.dot(x2[:, PW:], wbhi_ref[...],
                          preferred_element_type=jnp.float32)
        h1 = jnp.maximum(h1, 0.0)
        h1s_ref[...] = h1.reshape(G, 16, F).sum(axis=1)
        xlo = flo.reshape(G, 16, PW).sum(axis=1)
        xhi = fhi.reshape(G, 16, PW).sum(axis=1)
        f0lo, f0hi = _unpack_feat(f0_ref[...])
        h0 = jnp.dot(f0lo, walo_ref[...], preferred_element_type=jnp.float32)
        h0 = h0 + jnp.dot(f0hi, wahi_ref[...], preferred_element_type=jnp.float32)
        h0 = h0 + jnp.dot(xlo, wblo_ref[...], preferred_element_type=jnp.float32)
        h0 = h0 + jnp.dot(xhi, wbhi_ref[...], preferred_element_type=jnp.float32)
        h0_ref[...] = jnp.maximum(h0, 0.0)

    return pl.pallas_call(
        body,
        grid=(grid,),
        in_specs=[
            pl.BlockSpec((R, PW), lambda i: (i, 0)),
            pl.BlockSpec((R, F), lambda i: (i, 0)),
            pl.BlockSpec((G, PW), lambda i: (i, 0)),
            pl.BlockSpec((PW, F), lambda i: (0, 0)),
            pl.BlockSpec((PW, F), lambda i: (0, 0)),
            pl.BlockSpec((PW, F), lambda i: (0, 0)),
            pl.BlockSpec((PW, F), lambda i: (0, 0)),
        ],
        out_specs=[
            pl.BlockSpec((G, F), lambda i: (i, 0)),
            pl.BlockSpec((G, F), lambda i: (i, 0)),
        ],
        out_shape=[
            jax.ShapeDtypeStruct((B, F), jnp.float32),
            jax.ShapeDtypeStruct((B, F), jnp.float32),
        ],
    )(feat1p, x2s, feat0p, walo, wahi, wb16lo, wb16hi)


def _tc_layer2(h0, h1s, w2a, w2b16):
    B, H = h0.shape

    def body(h0_ref, h1_ref, wa_ref, wb_ref, out_ref):
        o = jnp.dot(h0_ref[...], wa_ref[...], preferred_element_type=jnp.float32)
        o = o + jnp.dot(h1_ref[...], wb_ref[...], preferred_element_type=jnp.float32)
        out_ref[...] = jnp.maximum(o, 0.0)

    return pl.pallas_call(
        body,
        out_shape=jax.ShapeDtypeStruct((B, H), jnp.float32),
    )(h0, h1s, w2a, w2b16)


def kernel(forest0, forest1, forest2, feature_matrix, W1, W2):
    N, F = feature_matrix.shape
    H = F // 2
    f0 = forest0.astype(jnp.int32)
    f1 = forest1.reshape(-1).astype(jnp.int32)
    f2 = forest2.reshape(-1).astype(jnp.int32)

    fmp = _tc_pack(feature_matrix)

    feat0p, feat1p, x2s = _sc_gather_all(f0, f1.reshape(-1, 128), f2.reshape(-1, 128), fmp)

    W1t = W1.T
    w1a = W1t[:F]
    w1b16 = W1t[F:] * (1.0 / 16.0)
    walo, wahi = w1a[:H], w1a[H:]
    wb16lo, wb16hi = w1b16[:H], w1b16[H:]

    W2t = W2.T
    w2a = W2t[:F]
    w2b16 = W2t[F:] * (1.0 / 16.0)

    h0, h1s = _tc_layer1(feat1p, x2s, feat0p, walo, wahi, wb16lo, wb16hi)
    return _tc_layer2(h0, h1s, w2a, w2b16)
